# baseline (speedup 1.0000x reference)
import jax
import jax.numpy as jnp
from jax import lax
from jax.experimental import pallas as pl
from jax.experimental.pallas import tpu as pltpu
from jax.experimental.pallas import tpu_sc as plsc

_BATCH = 4096
_FIELDS = 26
_LANES = 16
_NUM_CORES = 2
_NUM_SUBCORES = 16
_NUM_WORKERS = _NUM_CORES * _NUM_SUBCORES
_BPW = _BATCH // _NUM_WORKERS
_CHUNKS = _BPW // _LANES
_SLAB = _BPW * _FIELDS
_VROWS = 2600000
_HALF = _VROWS // 2


def _wid():
    return lax.axis_index("s") * _NUM_CORES + lax.axis_index("c")


def _fire_and_drain(src_hbm, idx2d, val2d, sem):
    copies = []
    for f in range(_FIELDS):
        cp = pltpu.make_async_copy(src_hbm.at[idx2d.at[f]], val2d.at[f], sem)
        cp.start()
        copies.append(cp)
    for cp in copies:
        cp.wait()


def _body_lo(x_hbm, off_hbm, dlo_hbm, part_hbm, idx_hbm,
             xv, off_v, idx2d, idxc2d, val2d, acc_v, sem_g):
    w = _wid()
    base = w * _BPW

    pltpu.sync_copy(x_hbm.at[pl.ds(base * _FIELDS, _SLAB)], xv)
    pltpu.sync_copy(off_hbm, off_v)

    iota26 = lax.iota(jnp.int32, _LANES) * _FIELDS
    half_m1 = jnp.full((_LANES,), _HALF - 1, jnp.int32)

    for f in range(_FIELDS):
        off_b = off_v[pl.ds(f * _LANES, _LANES)]
        for c in range(_CHUNKS):
            iv = iota26 + (c * _LANES * _FIELDS + f)
            v = plsc.load_gather(xv, [iv]) + off_b
            idx2d[f, pl.ds(c * _LANES, _LANES)] = v
            idxc2d[f, pl.ds(c * _LANES, _LANES)] = jnp.minimum(v, half_m1)

    pltpu.sync_copy(idx2d, idx_hbm.at[w])

    _fire_and_drain(dlo_hbm, idxc2d, val2d, sem_g)

    zero = jnp.zeros((_LANES,), jnp.float32)
    half = jnp.full((_LANES,), _HALF, jnp.int32)
    for c in range(_CHUNKS):
        acc = zero
        for f in range(_FIELDS):
            iv = idx2d[f, pl.ds(c * _LANES, _LANES)]
            vv = val2d[f, pl.ds(c * _LANES, _LANES)]
            acc = acc + jnp.where(iv < half, vv, zero)
        acc_v[pl.ds(c * _LANES, _LANES)] = acc

    pltpu.sync_copy(acc_v, part_hbm.at[pl.ds(base, _BPW)])


def _body_hi(idx_all_hbm, part_hbm, bias_hbm, dhi_hbm, out_hbm,
             idx2d, idxc2d, val2d, bias_v, part_v, acc_v, sem_g):
    w = _wid()
    base = w * _BPW

    pltpu.sync_copy(idx_all_hbm.at[w], idx2d)
    pltpu.sync_copy(part_hbm.at[pl.ds(base, _BPW)], part_v)
    pltpu.sync_copy(bias_hbm, bias_v)

    half = jnp.full((_LANES,), _HALF, jnp.int32)
    zero_i = jnp.zeros((_LANES,), jnp.int32)
    for f in range(_FIELDS):
        for c in range(_CHUNKS):
            v = idx2d[f, pl.ds(c * _LANES, _LANES)]
            idxc2d[f, pl.ds(c * _LANES, _LANES)] = jnp.maximum(
                v - half, zero_i)

    _fire_and_drain(dhi_hbm, idxc2d, val2d, sem_g)

    zero = jnp.zeros((_LANES,), jnp.float32)
    bias_vec = bias_v[...]
    for c in range(_CHUNKS):
        acc = part_v[pl.ds(c * _LANES, _LANES)] + bias_vec
        for f in range(_FIELDS):
            iv = idx2d[f, pl.ds(c * _LANES, _LANES)]
            vv = val2d[f, pl.ds(c * _LANES, _LANES)]
            acc = acc + jnp.where(iv < half, zero, vv)
        acc_v[pl.ds(c * _LANES, _LANES)] = acc

    pltpu.sync_copy(acc_v, out_hbm.at[pl.ds(base, _BPW)])


@jax.jit
def kernel(x, table, bias, offsets):
    x_flat = x.reshape(-1)
    bias_b = jnp.broadcast_to(bias.astype(jnp.float32), (_LANES,))
    off_flat = jnp.broadcast_to(
        offsets.astype(jnp.int32)[:, None],
        (_FIELDS, _LANES)).reshape(-1)
    d_lo = table[:_HALF].reshape(_HALF)
    d_hi = table[_HALF:].reshape(_VROWS - _HALF)
    mesh = plsc.VectorSubcoreMesh(core_axis_name="c", subcore_axis_name="s",
                                  num_cores=_NUM_CORES,
                                  num_subcores=_NUM_SUBCORES)
    params = pltpu.CompilerParams(needs_layout_passes=False)

    run_lo = pl.kernel(
        _body_lo,
        out_type=(
            jax.ShapeDtypeStruct((_BATCH,), jnp.float32),
            jax.ShapeDtypeStruct((_NUM_WORKERS, _FIELDS, _BPW), jnp.int32),
        ),
        mesh=mesh,
        compiler_params=params,
        scratch_types=[
            pltpu.VMEM((_SLAB,), jnp.int32),
            pltpu.VMEM((_FIELDS * _LANES,), jnp.int32),
            pltpu.VMEM((_FIELDS, _BPW), jnp.int32),
            pltpu.VMEM((_FIELDS, _BPW), jnp.int32),
            pltpu.VMEM((_FIELDS, _BPW), jnp.float32),
            pltpu.VMEM((_BPW,), jnp.float32),
            pltpu.SemaphoreType.DMA,
        ],
    )
    part, idx_all = run_lo(x_flat, off_flat, d_lo)

    run_hi = pl.kernel(
        _body_hi,
        out_type=jax.ShapeDtypeStruct((_BATCH,), jnp.float32),
        mesh=mesh,
        compiler_params=params,
        scratch_types=[
            pltpu.VMEM((_FIELDS, _BPW), jnp.int32),
            pltpu.VMEM((_FIELDS, _BPW), jnp.int32),
            pltpu.VMEM((_FIELDS, _BPW), jnp.float32),
            pltpu.VMEM((_LANES,), jnp.float32),
            pltpu.VMEM((_BPW,), jnp.float32),
            pltpu.VMEM((_BPW,), jnp.float32),
            pltpu.SemaphoreType.DMA,
        ],
    )
    out = run_hi(idx_all, part, bias_b, d_hi)
    return out.reshape(_BATCH, 1)


# speedup vs baseline: 4.3566x; 4.3566x over previous
import jax
import jax.numpy as jnp
from jax import lax
from jax.experimental import pallas as pl
from jax.experimental.pallas import tpu as pltpu
from jax.experimental.pallas import tpu_sc as plsc

_BATCH = 4096
_FIELDS = 26
_LANES = 16
_NUM_CORES = 2
_NUM_SUBCORES = 16
_NUM_WORKERS = _NUM_CORES * _NUM_SUBCORES
_BPW = _BATCH // _NUM_WORKERS
_CHUNKS = _BPW // _LANES


def _body(xt_hbm, off_hbm, bias_hbm, table_hbm, out_hbm,
          xv, off_v, bias_v, idx2d, val2d, acc_v, sem_x, sem_g):
    wid = lax.axis_index("s") * _NUM_CORES + lax.axis_index("c")
    base = wid * _BPW

    cpx = pltpu.make_async_copy(xt_hbm.at[:, pl.ds(base, _BPW)], xv, sem_x)
    cpx.start()
    pltpu.sync_copy(off_hbm, off_v)
    pltpu.sync_copy(bias_hbm, bias_v)
    cpx.wait()

    gathers = []
    for f in range(_FIELDS):
        off_b = off_v[f, :]
        for c in range(_CHUNKS):
            sl = pl.ds(c * _LANES, _LANES)
            idx2d[f, sl] = xv[f, sl] + off_b
        cp = pltpu.make_async_copy(table_hbm.at[idx2d.at[f]], val2d.at[f],
                                   sem_g)
        cp.start()
        gathers.append(cp)
    for cp in gathers:
        cp.wait()

    bias_vec = bias_v[...]
    accs = [bias_vec] * _CHUNKS
    for f in range(_FIELDS):
        for c in range(_CHUNKS):
            accs[c] = accs[c] + val2d[f, pl.ds(c * _LANES, _LANES)]
    for c in range(_CHUNKS):
        acc_v[pl.ds(c * _LANES, _LANES)] = accs[c]

    pltpu.sync_copy(acc_v, out_hbm.at[pl.ds(base, _BPW)])


@jax.jit
def kernel(x, table, bias, offsets):
    x_t = x.T
    table_flat = table.reshape(-1)
    bias_b = jnp.broadcast_to(bias.astype(jnp.float32), (_LANES,))
    off_b2d = jnp.broadcast_to(
        offsets.astype(jnp.int32)[:, None], (_FIELDS, _LANES))
    mesh = plsc.VectorSubcoreMesh(core_axis_name="c", subcore_axis_name="s",
                                  num_cores=_NUM_CORES,
                                  num_subcores=_NUM_SUBCORES)
    run = pl.kernel(
        _body,
        out_type=jax.ShapeDtypeStruct((_BATCH,), jnp.float32),
        mesh=mesh,
        compiler_params=pltpu.CompilerParams(needs_layout_passes=False),
        scratch_types=[
            pltpu.VMEM((_FIELDS, _BPW), jnp.int32),
            pltpu.VMEM((_FIELDS, _LANES), jnp.int32),
            pltpu.VMEM((_LANES,), jnp.float32),
            pltpu.VMEM((_FIELDS, _BPW), jnp.int32),
            pltpu.VMEM((_FIELDS, _BPW), jnp.float32),
            pltpu.VMEM((_BPW,), jnp.float32),
            pltpu.SemaphoreType.DMA,
            pltpu.SemaphoreType.DMA,
        ],
    )
    out = run(x_t, off_b2d, bias_b, table_flat)
    return out.reshape(_BATCH, 1)


# speedup vs baseline: 4.3607x; 1.0010x over previous
import jax
import jax.numpy as jnp
from jax import lax
from jax.experimental import pallas as pl
from jax.experimental.pallas import tpu as pltpu
from jax.experimental.pallas import tpu_sc as plsc

_BATCH = 4096
_FIELDS = 26
_LANES = 16
_NUM_CORES = 2
_NUM_SUBCORES = 16
_NUM_WORKERS = _NUM_CORES * _NUM_SUBCORES
_BPW = _BATCH // _NUM_WORKERS
_CHUNKS = _BPW // _LANES


def _body(xt_hbm, off_hbm, bias_hbm, table_hbm, out_hbm,
          xv, off_v, bias_v, idx1d, val1d, acc_v, sem_x, sem_g):
    wid = lax.axis_index("s") * _NUM_CORES + lax.axis_index("c")
    base = wid * _BPW

    cpx = pltpu.make_async_copy(xt_hbm.at[:, pl.ds(base, _BPW)], xv, sem_x)
    cpx.start()
    pltpu.sync_copy(off_hbm, off_v)
    pltpu.sync_copy(bias_hbm, bias_v)
    cpx.wait()

    for f in range(_FIELDS):
        off_b = off_v[f, :]
        for c in range(_CHUNKS):
            idx1d[pl.ds(f * _BPW + c * _LANES, _LANES)] = (
                xv[f, pl.ds(c * _LANES, _LANES)] + off_b)
    cp = pltpu.make_async_copy(table_hbm.at[idx1d], val1d, sem_g)
    cp.start()
    cp.wait()

    bias_vec = bias_v[...]
    accs = [bias_vec] * _CHUNKS
    for f in range(_FIELDS):
        for c in range(_CHUNKS):
            accs[c] = accs[c] + val1d[pl.ds(f * _BPW + c * _LANES, _LANES)]
    for c in range(_CHUNKS):
        acc_v[pl.ds(c * _LANES, _LANES)] = accs[c]

    pltpu.sync_copy(acc_v, out_hbm.at[pl.ds(base, _BPW)])


@jax.jit
def kernel(x, table, bias, offsets):
    x_t = x.T
    table_flat = table.reshape(-1)
    bias_b = jnp.broadcast_to(bias.astype(jnp.float32), (_LANES,))
    off_b2d = jnp.broadcast_to(
        offsets.astype(jnp.int32)[:, None], (_FIELDS, _LANES))
    mesh = plsc.VectorSubcoreMesh(core_axis_name="c", subcore_axis_name="s",
                                  num_cores=_NUM_CORES,
                                  num_subcores=_NUM_SUBCORES)
    run = pl.kernel(
        _body,
        out_type=jax.ShapeDtypeStruct((_BATCH,), jnp.float32),
        mesh=mesh,
        compiler_params=pltpu.CompilerParams(needs_layout_passes=False),
        scratch_types=[
            pltpu.VMEM((_FIELDS, _BPW), jnp.int32),
            pltpu.VMEM((_FIELDS, _LANES), jnp.int32),
            pltpu.VMEM((_LANES,), jnp.float32),
            pltpu.VMEM((_FIELDS * _BPW,), jnp.int32),
            pltpu.VMEM((_FIELDS * _BPW,), jnp.float32),
            pltpu.VMEM((_BPW,), jnp.float32),
            pltpu.SemaphoreType.DMA,
            pltpu.SemaphoreType.DMA,
        ],
    )
    out = run(x_t, off_b2d, bias_b, table_flat)
    return out.reshape(_BATCH, 1)


# speedup vs baseline: 4.3764x; 1.0036x over previous
import jax
import jax.numpy as jnp
from jax import lax
from jax.experimental import pallas as pl
from jax.experimental.pallas import tpu as pltpu
from jax.experimental.pallas import tpu_sc as plsc

_BATCH = 4096
_FIELDS = 26
_LANES = 16
_NUM_CORES = 2
_NUM_SUBCORES = 16
_NUM_WORKERS = _NUM_CORES * _NUM_SUBCORES
_BPW = _BATCH // _NUM_WORKERS
_CHUNKS = _BPW // _LANES
_SLAB = _FIELDS * _BPW


def _wid():
    return lax.axis_index("s") * _NUM_CORES + lax.axis_index("c")


def _body_idx(xt_hbm, off_hbm, idx_hbm, xv, off_v, idx1d, sem_x):
    w = _wid()
    base = w * _BPW

    cpx = pltpu.make_async_copy(xt_hbm.at[:, pl.ds(base, _BPW)], xv, sem_x)
    cpx.start()
    pltpu.sync_copy(off_hbm, off_v)
    cpx.wait()

    for f in range(_FIELDS):
        off_b = off_v[f, :]
        for c in range(_CHUNKS):
            idx1d[pl.ds(f * _BPW + c * _LANES, _LANES)] = (
                xv[f, pl.ds(c * _LANES, _LANES)] + off_b)

    pltpu.sync_copy(idx1d, idx_hbm.at[w])


def _body_gather(idx_hbm, bias_hbm, table_hbm, out_hbm,
                 idx1d, val1d, bias_v, acc_v, sem_g):
    w = _wid()
    base = w * _BPW

    pltpu.sync_copy(idx_hbm.at[w], idx1d)
    pltpu.sync_copy(bias_hbm, bias_v)

    cp = pltpu.make_async_copy(table_hbm.at[idx1d], val1d, sem_g)
    cp.start()
    cp.wait()

    bias_vec = bias_v[...]
    accs = [bias_vec] * _CHUNKS
    for f in range(_FIELDS):
        for c in range(_CHUNKS):
            accs[c] = accs[c] + val1d[pl.ds(f * _BPW + c * _LANES, _LANES)]
    for c in range(_CHUNKS):
        acc_v[pl.ds(c * _LANES, _LANES)] = accs[c]

    pltpu.sync_copy(acc_v, out_hbm.at[pl.ds(base, _BPW)])


@jax.jit
def kernel(x, table, bias, offsets):
    bias_b = jnp.broadcast_to(bias.astype(jnp.float32), (_LANES,))
    off_b2d = jnp.broadcast_to(
        offsets.astype(jnp.int32)[:, None], (_FIELDS, _LANES))
    x_t = x.T
    table_flat = table.reshape(-1)
    mesh = plsc.VectorSubcoreMesh(core_axis_name="c", subcore_axis_name="s",
                                  num_cores=_NUM_CORES,
                                  num_subcores=_NUM_SUBCORES)
    params = pltpu.CompilerParams(needs_layout_passes=False)

    run_idx = pl.kernel(
        _body_idx,
        out_type=jax.ShapeDtypeStruct((_NUM_WORKERS, _SLAB), jnp.int32),
        mesh=mesh,
        compiler_params=params,
        scratch_types=[
            pltpu.VMEM((_FIELDS, _BPW), jnp.int32),
            pltpu.VMEM((_FIELDS, _LANES), jnp.int32),
            pltpu.VMEM((_SLAB,), jnp.int32),
            pltpu.SemaphoreType.DMA,
        ],
    )
    idx_all = run_idx(x_t, off_b2d)

    run_gather = pl.kernel(
        _body_gather,
        out_type=jax.ShapeDtypeStruct((_BATCH,), jnp.float32),
        mesh=mesh,
        compiler_params=params,
        scratch_types=[
            pltpu.VMEM((_SLAB,), jnp.int32),
            pltpu.VMEM((_SLAB,), jnp.float32),
            pltpu.VMEM((_LANES,), jnp.float32),
            pltpu.VMEM((_BPW,), jnp.float32),
            pltpu.SemaphoreType.DMA,
        ],
    )
    out = run_gather(idx_all, bias_b, table_flat)
    return out.reshape(_BATCH, 1)
